# baseline (speedup 1.0000x reference)
import functools

import jax
import jax.numpy as jnp
from jax import lax
from jax.experimental import pallas as pl
from jax.experimental.pallas import tpu as pltpu
from jax.experimental.pallas import tpu_sc as plsc

NN = 10000
EE = 320000
DD = 128
NG = 16
EPS = 1e-5

NND = 10112
NW = 32
CH = 128
NCH = 84
EPT = NCH * CH
EPAD = NW * EPT
NRB = 3
IBN = 4
ROWS_PT = NND // 16

_mesh = plsc.VectorSubcoreMesh(core_axis_name="c", subcore_axis_name="s")


def _zero_acc_slice(rows0, acc, sid):
    zero = jnp.zeros((16,), jnp.float32)

    def _zrow(i, _):
        for j in range(DD // 16):
            rows0[i, pl.ds(j * 16, 16)] = zero
        return 0

    lax.fori_loop(0, CH, _zrow, 0)
    rbase = sid * ROWS_PT
    for r in range(ROWS_PT // CH):
        pltpu.sync_copy(rows0, acc.at[pl.ds(rbase + r * CH, CH)])
    rem = ROWS_PT % CH
    if rem:
        pltpu.sync_copy(rows0.at[pl.ds(0, rem)],
                        acc.at[pl.ds(rbase + (ROWS_PT // CH) * CH, rem)])


def _writeback(acc, out_hbm, cid, sid):
    rbase = sid * ROWS_PT
    for r in range(ROWS_PT // CH):
        sl = pl.ds(rbase + r * CH, CH)
        pltpu.sync_copy(acc.at[sl], out_hbm.at[cid, sl])
    rem = ROWS_PT % CH
    if rem:
        sl = pl.ds(rbase + (ROWS_PT // CH) * CH, rem)
        pltpu.sync_copy(acc.at[sl], out_hbm.at[cid, sl])


@functools.partial(
    pl.kernel,
    out_type=jax.ShapeDtypeStruct((2, NND, DD), jnp.float32),
    mesh=_mesh,
    scratch_types=[
        [pltpu.VMEM((2, CH), jnp.int32) for _ in range(IBN)],
        [pltpu.VMEM((CH, DD), jnp.float32) for _ in range(NRB)],
        pltpu.VMEM_SHARED((NND, DD), jnp.float32),
        [pltpu.SemaphoreType.DMA for _ in range(IBN)],
        [pltpu.SemaphoreType.DMA for _ in range(NRB)],
    ],
)
def _agg(g_hbm, eidx_hbm, out_hbm, ibufs, rows, acc, isems, gsems):
    cid = lax.axis_index("c")
    sid = lax.axis_index("s")
    wid = cid * 16 + sid

    for i in range(IBN):
        pltpu.async_copy(eidx_hbm.at[wid, i], ibufs[i], isems[i])
    _zero_acc_slice(rows[0], acc, sid)
    plsc.subcore_barrier()
    pltpu.make_async_copy(eidx_hbm.at[wid, 0], ibufs[0], isems[0]).wait()
    pltpu.make_async_copy(eidx_hbm.at[wid, 0], ibufs[1], isems[1]).wait()
    pltpu.async_copy(g_hbm.at[ibufs[0].at[0]], rows[0], gsems[0])
    pltpu.async_copy(g_hbm.at[ibufs[1].at[0]], rows[1], gsems[1])

    def _body(t, _):
        for i in range(12):
            k = 12 * t + i
            b = i % NRB
            s4 = i % IBN
            n4 = (i + 2) % IBN
            pltpu.make_async_copy(g_hbm.at[ibufs[0].at[0]], rows[b],
                                  gsems[b]).wait()
            pltpu.make_async_copy(eidx_hbm.at[wid, 0], ibufs[n4],
                                  isems[n4]).wait()
            pltpu.async_copy(g_hbm.at[ibufs[n4].at[0]],
                             rows[(i + 2) % NRB], gsems[(i + 2) % NRB])
            pltpu.sync_copy(rows[b], acc.at[ibufs[s4].at[1]], add=True)
            pltpu.async_copy(eidx_hbm.at[wid, k + IBN], ibufs[s4], isems[s4])
        return 0

    lax.fori_loop(0, NCH // 12, _body, 0)
    pltpu.make_async_copy(g_hbm.at[ibufs[0].at[0]], rows[NCH % NRB],
                          gsems[NCH % NRB]).wait()
    pltpu.make_async_copy(g_hbm.at[ibufs[0].at[0]], rows[(NCH + 1) % NRB],
                          gsems[(NCH + 1) % NRB]).wait()
    pltpu.make_async_copy(eidx_hbm.at[wid, 0], ibufs[(NCH + 2) % IBN],
                          isems[(NCH + 2) % IBN]).wait()
    pltpu.make_async_copy(eidx_hbm.at[wid, 0], ibufs[(NCH + 3) % IBN],
                          isems[(NCH + 3) % IBN]).wait()
    plsc.subcore_barrier()
    _writeback(acc, out_hbm, cid, sid)


def _mm_body(x_ref, w_ref, o_ref):
    o_ref[...] = jnp.dot(x_ref[...], w_ref[...],
                         preferred_element_type=jnp.float32)


_mm = pl.pallas_call(
    _mm_body,
    out_shape=jax.ShapeDtypeStruct((NND, DD), jnp.float32),
)


def _scale_body(d0_ref, d1_ref, m_ref, g_ref, dinv_ref):
    deg = d0_ref[...] + d1_ref[...] + 1.0
    dinv = lax.rsqrt(deg)
    dinv_ref[...] = dinv
    g_ref[...] = dinv * m_ref[...]


_scale = pl.pallas_call(
    _scale_body,
    out_shape=(
        jax.ShapeDtypeStruct((NND, DD), jnp.float32),
        jax.ShapeDtypeStruct((NND, 1), jnp.float32),
    ),
)


def _norm_body(p0_ref, p1_ref, g_ref, dinv_ref, b_ref,
               al_ref, ga_ref, be_ref, w_ref, o_ref):
    dinv = dinv_ref[...]
    h = dinv * (p0_ref[...] + p1_ref[...] + g_ref[...]) + b_ref[...]
    rmask = lax.broadcasted_iota(jnp.int32, (NND, 1), 0) < NN
    h = jnp.where(rmask, h, 0.0)
    mean = jnp.sum(h, axis=0, keepdims=True) * (1.0 / NN)
    o = h - al_ref[...] * mean
    o = jnp.where(rmask, o, 0.0)
    var = jnp.sum(o * o, axis=0, keepdims=True) * (1.0 / NN)
    a = ga_ref[...] * (o * lax.rsqrt(var + EPS)) + be_ref[...]
    a = jnp.where(rmask, jnp.maximum(a, 0.0), 0.0)
    o_ref[...] = dinv * jnp.dot(a, w_ref[...],
                                preferred_element_type=jnp.float32)


_norm = pl.pallas_call(
    _norm_body,
    out_shape=jax.ShapeDtypeStruct((NND, DD), jnp.float32),
)


def _head_body(p0_ref, p1_ref, g_ref, dinv_ref, b_ref, batch_ref,
               wh1_ref, bh1_ref, wh2_ref, bh2_ref, o_ref):
    h = dinv_ref[...] * (p0_ref[...] + p1_ref[...] + g_ref[...]) + b_ref[...]
    gid = lax.broadcasted_iota(jnp.int32, (NG, NND), 0)
    oh = (gid == batch_ref[...]).astype(jnp.float32)
    sums = jnp.dot(oh, h, preferred_element_type=jnp.float32)
    cnt = jnp.sum(oh, axis=1, keepdims=True)
    pooled = sums / jnp.maximum(cnt, 1.0)
    z = jnp.maximum(
        jnp.dot(pooled, wh1_ref[...], preferred_element_type=jnp.float32)
        + bh1_ref[...], 0.0)
    o_ref[...] = jnp.dot(z, wh2_ref[...],
                         preferred_element_type=jnp.float32) + bh2_ref[...]


def _make_head(nhid, nout):
    return pl.pallas_call(
        _head_body,
        out_shape=jax.ShapeDtypeStruct((NG, nout), jnp.float32),
    )


@jax.jit
def kernel(x, edge_index, batch, params):
    src = edge_index[0].astype(jnp.int32)
    dst = edge_index[1].astype(jnp.int32)
    pad = jnp.full((EPAD - EE,), NN, dtype=jnp.int32)
    srcp = jnp.concatenate([src, pad]).reshape(NW, NCH, CH)
    dstp = jnp.concatenate([dst, pad]).reshape(NW, NCH, CH)
    eidx = jnp.stack([srcp, dstp], axis=2)
    eidx = jnp.pad(eidx, ((0, 0), (0, IBN), (0, 0), (0, 0)),
                   constant_values=NN)

    xp = jnp.pad(x, ((0, NND - NN), (0, 0)))
    batchp = jnp.pad(batch.astype(jnp.int32), (0, NND - NN),
                     constant_values=NG).reshape(1, NND)

    def row(v):
        return v.reshape(1, -1)

    ones_g = jnp.zeros((NND, DD), jnp.float32).at[:NN].set(1.0)
    degp = _agg(ones_g, eidx)
    m1 = _mm(xp, params['W1'])
    g1, dinv = _scale(degp[0, :, 0:1], degp[1, :, 0:1], m1)

    a1 = _agg(g1, eidx)
    g2 = _norm(a1[0], a1[1], g1, dinv, row(params['b1']),
               row(params['alpha1']), row(params['gamma1']),
               row(params['beta1']), params['W2'])

    a2 = _agg(g2, eidx)
    g3 = _norm(a2[0], a2[1], g2, dinv, row(params['b2']),
               row(params['alpha2']), row(params['gamma2']),
               row(params['beta2']), params['W3'])

    a3 = _agg(g3, eidx)
    head = _make_head(params['Wh1'].shape[1], params['Wh2'].shape[1])
    return head(a3[0], a3[1], g3, dinv, row(params['b3']), batchp,
                params['Wh1'], row(params['bh1']),
                params['Wh2'], row(params['bh2']))


# speedup vs baseline: 12.8797x; 12.8797x over previous
import functools

import jax
import jax.numpy as jnp
from jax import lax
from jax.experimental import pallas as pl
from jax.experimental.pallas import tpu as pltpu
from jax.experimental.pallas import tpu_sc as plsc

NN = 10000
EE = 320000
DD = 128
NG = 16
EPS = 1e-5

NND = 10112
NW = 32
CH = 128
NCH = 84
EPT = NCH * CH
EPAD = NW * EPT
NRB = 3
IBN = 4
ROWS_PT = NND // 16

_mesh = plsc.VectorSubcoreMesh(core_axis_name="c", subcore_axis_name="s")


def _zero_acc_slice(rows0, acc, sid):
    zero = jnp.zeros((16,), jnp.float32)

    def _zrow(i, _):
        for j in range(DD // 16):
            rows0[i, pl.ds(j * 16, 16)] = zero
        return 0

    lax.fori_loop(0, CH, _zrow, 0)
    rbase = sid * ROWS_PT
    for r in range(ROWS_PT // CH):
        pltpu.sync_copy(rows0, acc.at[pl.ds(rbase + r * CH, CH)])
    rem = ROWS_PT % CH
    if rem:
        pltpu.sync_copy(rows0.at[pl.ds(0, rem)],
                        acc.at[pl.ds(rbase + (ROWS_PT // CH) * CH, rem)])


def _writeback(acc, out_hbm, cid, sid):
    rbase = sid * ROWS_PT
    for r in range(ROWS_PT // CH):
        sl = pl.ds(rbase + r * CH, CH)
        pltpu.sync_copy(acc.at[sl], out_hbm.at[cid, sl])
    rem = ROWS_PT % CH
    if rem:
        sl = pl.ds(rbase + (ROWS_PT // CH) * CH, rem)
        pltpu.sync_copy(acc.at[sl], out_hbm.at[cid, sl])


@functools.partial(
    pl.kernel,
    out_type=jax.ShapeDtypeStruct((2, NND, DD), jnp.float32),
    mesh=_mesh,
    scratch_types=[
        [pltpu.VMEM((2, CH), jnp.int32) for _ in range(IBN)],
        [pltpu.VMEM((CH, DD), jnp.float32) for _ in range(NRB)],
        pltpu.VMEM_SHARED((NND, DD), jnp.float32),
        [pltpu.SemaphoreType.DMA for _ in range(IBN)],
        [pltpu.SemaphoreType.DMA for _ in range(NRB)],
    ],
)
def _agg(g_hbm, eidx_hbm, out_hbm, ibufs, rows, acc, isems, gsems):
    cid = lax.axis_index("c")
    sid = lax.axis_index("s")
    wid = cid * 16 + sid

    for i in range(IBN):
        pltpu.async_copy(eidx_hbm.at[wid, i], ibufs[i], isems[i])
    _zero_acc_slice(rows[0], acc, sid)
    plsc.subcore_barrier()
    pltpu.make_async_copy(eidx_hbm.at[wid, 0], ibufs[0], isems[0]).wait()
    pltpu.make_async_copy(eidx_hbm.at[wid, 0], ibufs[1], isems[1]).wait()
    pltpu.async_copy(g_hbm.at[ibufs[0].at[0]], rows[0], gsems[0])
    pltpu.async_copy(g_hbm.at[ibufs[1].at[0]], rows[1], gsems[1])

    def _body(t, _):
        for i in range(12):
            k = 12 * t + i
            b = i % NRB
            s4 = i % IBN
            n4 = (i + 2) % IBN
            pltpu.make_async_copy(g_hbm.at[ibufs[0].at[0]], rows[b],
                                  gsems[b]).wait()
            pltpu.make_async_copy(eidx_hbm.at[wid, 0], ibufs[n4],
                                  isems[n4]).wait()
            pltpu.async_copy(g_hbm.at[ibufs[n4].at[0]],
                             rows[(i + 2) % NRB], gsems[(i + 2) % NRB])
            pltpu.sync_copy(rows[b], acc.at[ibufs[s4].at[1]], add=True)
            pltpu.async_copy(eidx_hbm.at[wid, k + IBN], ibufs[s4], isems[s4])
        return 0

    lax.fori_loop(0, NCH // 12, _body, 0)
    pltpu.make_async_copy(g_hbm.at[ibufs[0].at[0]], rows[NCH % NRB],
                          gsems[NCH % NRB]).wait()
    pltpu.make_async_copy(g_hbm.at[ibufs[0].at[0]], rows[(NCH + 1) % NRB],
                          gsems[(NCH + 1) % NRB]).wait()
    pltpu.make_async_copy(eidx_hbm.at[wid, 0], ibufs[(NCH + 2) % IBN],
                          isems[(NCH + 2) % IBN]).wait()
    pltpu.make_async_copy(eidx_hbm.at[wid, 0], ibufs[(NCH + 3) % IBN],
                          isems[(NCH + 3) % IBN]).wait()
    plsc.subcore_barrier()
    _writeback(acc, out_hbm, cid, sid)


def _mm_body(x_ref, w_ref, o_ref):
    o_ref[...] = jnp.dot(x_ref[...], w_ref[...],
                         preferred_element_type=jnp.float32)


_mm = pl.pallas_call(
    _mm_body,
    out_shape=jax.ShapeDtypeStruct((NND, DD), jnp.float32),
)


def _scale_body(d0_ref, d1_ref, m_ref, g_ref, dinv_ref):
    deg = d0_ref[...] + d1_ref[...] + 1.0
    dinv = lax.rsqrt(deg)
    dinv_ref[...] = dinv
    g_ref[...] = dinv * m_ref[...]


_scale = pl.pallas_call(
    _scale_body,
    out_shape=(
        jax.ShapeDtypeStruct((NND, DD), jnp.float32),
        jax.ShapeDtypeStruct((NND, 1), jnp.float32),
    ),
)


def _norm_body(p0_ref, p1_ref, g_ref, dinv_ref, b_ref,
               al_ref, ga_ref, be_ref, w_ref, o_ref):
    dinv = dinv_ref[...]
    h = dinv * (p0_ref[...] + p1_ref[...] + g_ref[...]) + b_ref[...]
    rmask = lax.broadcasted_iota(jnp.int32, (NND, 1), 0) < NN
    h = jnp.where(rmask, h, 0.0)
    mean = jnp.sum(h, axis=0, keepdims=True) * (1.0 / NN)
    o = h - al_ref[...] * mean
    o = jnp.where(rmask, o, 0.0)
    var = jnp.sum(o * o, axis=0, keepdims=True) * (1.0 / NN)
    a = ga_ref[...] * (o * lax.rsqrt(var + EPS)) + be_ref[...]
    a = jnp.where(rmask, jnp.maximum(a, 0.0), 0.0)
    o_ref[...] = dinv * jnp.dot(a, w_ref[...],
                                preferred_element_type=jnp.float32)


_norm = pl.pallas_call(
    _norm_body,
    out_shape=jax.ShapeDtypeStruct((NND, DD), jnp.float32),
)


def _head_body(p0_ref, p1_ref, g_ref, dinv_ref, b_ref, batch_ref,
               wh1_ref, bh1_ref, wh2_ref, bh2_ref, o_ref):
    h = dinv_ref[...] * (p0_ref[...] + p1_ref[...] + g_ref[...]) + b_ref[...]
    gid = lax.broadcasted_iota(jnp.int32, (NG, NND), 0)
    oh = (gid == batch_ref[...]).astype(jnp.float32)
    sums = jnp.dot(oh, h, preferred_element_type=jnp.float32)
    cnt = jnp.sum(oh, axis=1, keepdims=True)
    pooled = sums / jnp.maximum(cnt, 1.0)
    z = jnp.maximum(
        jnp.dot(pooled, wh1_ref[...], preferred_element_type=jnp.float32)
        + bh1_ref[...], 0.0)
    o_ref[...] = jnp.dot(z, wh2_ref[...],
                         preferred_element_type=jnp.float32) + bh2_ref[...]


def _make_head(nhid, nout):
    return pl.pallas_call(
        _head_body,
        out_shape=jax.ShapeDtypeStruct((NG, nout), jnp.float32),
    )


@jax.jit
def kernel(x, edge_index, batch, params):
    src = edge_index[0].astype(jnp.int32)
    dst = edge_index[1].astype(jnp.int32)
    pe = jnp.arange(EPAD - EE, dtype=jnp.int32)
    srcp = jnp.concatenate([src, pe % NN]).reshape(NW, NCH, CH)
    dstp = jnp.concatenate([dst, NN + pe % (NND - NN)]).reshape(NW, NCH, CH)
    eidx = jnp.stack([srcp, dstp], axis=2)
    ov = jnp.arange(NW * IBN * CH, dtype=jnp.int32)
    ov_src = (ov % NN).reshape(NW, IBN, 1, CH)
    ov_dst = (NN + ov % (NND - NN)).reshape(NW, IBN, 1, CH)
    eidx = jnp.concatenate(
        [eidx, jnp.concatenate([ov_src, ov_dst], axis=2)], axis=1)

    xp = jnp.pad(x, ((0, NND - NN), (0, 0)))
    batchp = jnp.pad(batch.astype(jnp.int32), (0, NND - NN),
                     constant_values=NG).reshape(1, NND)

    def row(v):
        return v.reshape(1, -1)

    ones_g = jnp.zeros((NND, DD), jnp.float32).at[:NN].set(1.0)
    degp = _agg(ones_g, eidx)
    m1 = _mm(xp, params['W1'])
    g1, dinv = _scale(degp[0, :, 0:1], degp[1, :, 0:1], m1)

    a1 = _agg(g1, eidx)
    g2 = _norm(a1[0], a1[1], g1, dinv, row(params['b1']),
               row(params['alpha1']), row(params['gamma1']),
               row(params['beta1']), params['W2'])

    a2 = _agg(g2, eidx)
    g3 = _norm(a2[0], a2[1], g2, dinv, row(params['b2']),
               row(params['alpha2']), row(params['gamma2']),
               row(params['beta2']), params['W3'])

    a3 = _agg(g3, eidx)
    head = _make_head(params['Wh1'].shape[1], params['Wh2'].shape[1])
    return head(a3[0], a3[1], g3, dinv, row(params['b3']), batchp,
                params['Wh1'], row(params['bh1']),
                params['Wh2'], row(params['bh2']))
